# baseline (speedup 1.0000x reference)
import functools

import jax
import jax.numpy as jnp
from jax.experimental import pallas as pl

H = 16


def _qkv_softmax_kernel(x_ref, w_ref, b_ref, t_ref, *, scale):
    xb = x_ref[...]
    dn = (((1,), (1,)), ((), ()))
    q = jax.lax.dot_general(xb, w_ref[0, 0], dn,
                            preferred_element_type=jnp.float32) + b_ref[0, 0, 0]
    k = jax.lax.dot_general(xb, w_ref[1, 0], dn,
                            preferred_element_type=jnp.float32) + b_ref[1, 0, 0]
    v = jax.lax.dot_general(xb, w_ref[2, 0], dn,
                            preferred_element_type=jnp.float32) + b_ref[2, 0, 0]
    m = (q - k) * scale
    m = m - jnp.max(m, axis=-1, keepdims=True)
    e = jnp.exp(m)
    t_ref[...] = ((e / jnp.sum(e, axis=-1, keepdims=True)) * v).astype(jnp.bfloat16)


def _proj_kernel(t_ref, w_ref, b_ref, x_ref, o_ref):
    dn = (((1,), (1,)), ((), ()))
    o_ref[...] = (x_ref[...]
                  + jax.lax.dot_general(t_ref[...], w_ref[...], dn,
                                        preferred_element_type=jnp.float32)
                  + b_ref[0])


def kernel(x, Wqkv, bqkv, Wproj, bproj):
    B, S, D = x.shape
    dh = D // H
    scale = dh ** -0.5
    x2 = x.reshape(S, D)
    x_bf = x2.astype(jnp.bfloat16)
    w3 = Wqkv.reshape(3, H, dh, D).astype(jnp.bfloat16)
    b3 = bqkv.reshape(3, H, 1, dh)
    wp = Wproj.astype(jnp.bfloat16)

    t = pl.pallas_call(
        functools.partial(_qkv_softmax_kernel, scale=scale),
        grid=(H,),
        in_specs=[
            pl.BlockSpec((S, D), lambda h: (0, 0)),
            pl.BlockSpec((3, 1, dh, D), lambda h: (0, h, 0, 0)),
            pl.BlockSpec((3, 1, 1, dh), lambda h: (0, h, 0, 0)),
        ],
        out_specs=pl.BlockSpec((S, dh), lambda h: (0, h)),
        out_shape=jax.ShapeDtypeStruct((S, D), jnp.bfloat16),
    )(x_bf, w3, b3)

    BS2 = 1024
    out = pl.pallas_call(
        _proj_kernel,
        grid=(S // BS2,),
        in_specs=[
            pl.BlockSpec((BS2, D), lambda i: (i, 0)),
            pl.BlockSpec((D, D), lambda i: (0, 0)),
            pl.BlockSpec((1, D), lambda i: (0, 0)),
            pl.BlockSpec((BS2, D), lambda i: (i, 0)),
        ],
        out_specs=pl.BlockSpec((BS2, D), lambda i: (i, 0)),
        out_shape=jax.ShapeDtypeStruct((S, D), jnp.float32),
    )(t, wp, bproj.reshape(1, D), x2)

    return out.reshape(B, S, D)


# speedup vs baseline: 1.1136x; 1.1136x over previous
import functools

import jax
import jax.numpy as jnp
from jax.experimental import pallas as pl

H = 16


def _qkv_softmax_kernel(x_ref, w_ref, b_ref, t_ref, *, scale):
    xb = x_ref[...]
    dn = (((1,), (1,)), ((), ()))
    q = jax.lax.dot_general(xb, w_ref[0, 0].astype(jnp.bfloat16), dn,
                            preferred_element_type=jnp.float32) + b_ref[0, 0, 0]
    k = jax.lax.dot_general(xb, w_ref[1, 0].astype(jnp.bfloat16), dn,
                            preferred_element_type=jnp.float32) + b_ref[1, 0, 0]
    v = jax.lax.dot_general(xb, w_ref[2, 0].astype(jnp.bfloat16), dn,
                            preferred_element_type=jnp.float32) + b_ref[2, 0, 0]
    m = (q - k) * scale
    m = m - jnp.max(m, axis=-1, keepdims=True)
    e = jnp.exp(m)
    t_ref[...] = ((e / jnp.sum(e, axis=-1, keepdims=True)) * v).astype(jnp.bfloat16)


def _proj_kernel(t_ref, w_ref, b_ref, x_ref, o_ref):
    dn = (((1,), (1,)), ((), ()))
    o_ref[...] = (x_ref[...]
                  + jax.lax.dot_general(t_ref[...], w_ref[...].astype(jnp.bfloat16),
                                        dn, preferred_element_type=jnp.float32)
                  + b_ref[0])


def kernel(x, Wqkv, bqkv, Wproj, bproj):
    B, S, D = x.shape
    dh = D // H
    scale = dh ** -0.5
    x2 = x.reshape(S, D)
    x_bf = x2.astype(jnp.bfloat16)
    w3 = Wqkv.reshape(3, H, dh, D)
    b3 = bqkv.reshape(3, H, 1, dh)

    t = pl.pallas_call(
        functools.partial(_qkv_softmax_kernel, scale=scale),
        grid=(H,),
        in_specs=[
            pl.BlockSpec((S, D), lambda h: (0, 0)),
            pl.BlockSpec((3, 1, dh, D), lambda h: (0, h, 0, 0)),
            pl.BlockSpec((3, 1, 1, dh), lambda h: (0, h, 0, 0)),
        ],
        out_specs=pl.BlockSpec((S, dh), lambda h: (0, h)),
        out_shape=jax.ShapeDtypeStruct((S, D), jnp.bfloat16),
    )(x_bf, w3, b3)

    BS2 = 512
    out = pl.pallas_call(
        _proj_kernel,
        grid=(S // BS2,),
        in_specs=[
            pl.BlockSpec((BS2, D), lambda i: (i, 0)),
            pl.BlockSpec((D, D), lambda i: (0, 0)),
            pl.BlockSpec((1, D), lambda i: (0, 0)),
            pl.BlockSpec((BS2, D), lambda i: (i, 0)),
        ],
        out_specs=pl.BlockSpec((BS2, D), lambda i: (i, 0)),
        out_shape=jax.ShapeDtypeStruct((S, D), jnp.float32),
    )(t, Wproj, bproj.reshape(1, D), x2)

    return out.reshape(B, S, D)


# speedup vs baseline: 1.1572x; 1.0392x over previous
import functools

import jax
import jax.numpy as jnp
from jax.experimental import pallas as pl

H = 16


def _qkv_softmax_kernel(x_ref, w_ref, b_ref, t_ref, *, scale):
    xb = x_ref[...]
    dn = (((1,), (1,)), ((), ()))
    q = jax.lax.dot_general(xb, w_ref[0, 0], dn,
                            preferred_element_type=jnp.float32) + b_ref[0, 0, 0]
    k = jax.lax.dot_general(xb, w_ref[1, 0], dn,
                            preferred_element_type=jnp.float32) + b_ref[1, 0, 0]
    v = jax.lax.dot_general(xb, w_ref[2, 0], dn,
                            preferred_element_type=jnp.float32) + b_ref[2, 0, 0]
    m = (q - k) * scale
    m = m - jnp.max(m, axis=-1, keepdims=True)
    e = jnp.exp(m)
    t_ref[...] = (e / jnp.sum(e, axis=-1, keepdims=True)) * v


def _proj_kernel(t_ref, w_ref, b_ref, x_ref, o_ref):
    dn = (((1,), (1,)), ((), ()))
    o_ref[...] = (x_ref[...]
                  + jax.lax.dot_general(t_ref[...], w_ref[...], dn,
                                        preferred_element_type=jnp.float32)
                  + b_ref[0])


def kernel(x, Wqkv, bqkv, Wproj, bproj):
    B, S, D = x.shape
    dh = D // H
    scale = dh ** -0.5
    x2 = x.reshape(S, D)
    w3 = Wqkv.reshape(3, H, dh, D)
    b3 = bqkv.reshape(3, H, 1, dh)

    BS1 = 2048
    t = pl.pallas_call(
        functools.partial(_qkv_softmax_kernel, scale=scale),
        grid=(S // BS1, H),
        in_specs=[
            pl.BlockSpec((BS1, D), lambda i, h: (i, 0)),
            pl.BlockSpec((3, 1, dh, D), lambda i, h: (0, h, 0, 0)),
            pl.BlockSpec((3, 1, 1, dh), lambda i, h: (0, h, 0, 0)),
        ],
        out_specs=pl.BlockSpec((BS1, dh), lambda i, h: (i, h)),
        out_shape=jax.ShapeDtypeStruct((S, D), jnp.float32),
    )(x2, w3, b3)

    BS2 = 512
    out = pl.pallas_call(
        _proj_kernel,
        grid=(S // BS2,),
        in_specs=[
            pl.BlockSpec((BS2, D), lambda i: (i, 0)),
            pl.BlockSpec((D, D), lambda i: (0, 0)),
            pl.BlockSpec((1, D), lambda i: (0, 0)),
            pl.BlockSpec((BS2, D), lambda i: (i, 0)),
        ],
        out_specs=pl.BlockSpec((BS2, D), lambda i: (i, 0)),
        out_shape=jax.ShapeDtypeStruct((S, D), jnp.float32),
    )(t, Wproj, bproj.reshape(1, D), x2)

    return out.reshape(B, S, D)


# speedup vs baseline: 1.5216x; 1.3148x over previous
import functools

import jax
import jax.numpy as jnp
from jax.experimental import pallas as pl

H = 16


def _qkv_softmax_kernel(x_ref, w_ref, b_ref, t_ref, *, scale):
    xb = x_ref[...]
    dn = (((1,), (1,)), ((), ()))
    wd = (w_ref[0, 0] - w_ref[1, 0]) * scale
    m = (jax.lax.dot_general(xb, wd, dn, preferred_element_type=jnp.float32)
         + (b_ref[0, 0, 0] - b_ref[1, 0, 0]) * scale)
    v = jax.lax.dot_general(xb, w_ref[2, 0], dn,
                            preferred_element_type=jnp.float32) + b_ref[2, 0, 0]
    m = m - jnp.max(m, axis=-1, keepdims=True)
    e = jnp.exp(m)
    t_ref[...] = (e / jnp.sum(e, axis=-1, keepdims=True)) * v


def _proj_kernel(t_ref, w_ref, b_ref, x_ref, o_ref):
    dn = (((1,), (1,)), ((), ()))
    o_ref[...] = (x_ref[...]
                  + jax.lax.dot_general(t_ref[...], w_ref[...], dn,
                                        preferred_element_type=jnp.float32)
                  + b_ref[0])


def kernel(x, Wqkv, bqkv, Wproj, bproj):
    B, S, D = x.shape
    dh = D // H
    scale = dh ** -0.5
    x2 = x.reshape(S, D)
    w3 = Wqkv.reshape(3, H, dh, D)
    b3 = bqkv.reshape(3, H, 1, dh)

    BS1 = 2048
    t = pl.pallas_call(
        functools.partial(_qkv_softmax_kernel, scale=scale),
        grid=(S // BS1, H),
        in_specs=[
            pl.BlockSpec((BS1, D), lambda i, h: (i, 0)),
            pl.BlockSpec((3, 1, dh, D), lambda i, h: (0, h, 0, 0)),
            pl.BlockSpec((3, 1, 1, dh), lambda i, h: (0, h, 0, 0)),
        ],
        out_specs=pl.BlockSpec((BS1, dh), lambda i, h: (i, h)),
        out_shape=jax.ShapeDtypeStruct((S, D), jnp.float32),
    )(x2, w3, b3)

    BS2 = 512
    out = pl.pallas_call(
        _proj_kernel,
        grid=(S // BS2,),
        in_specs=[
            pl.BlockSpec((BS2, D), lambda i: (i, 0)),
            pl.BlockSpec((D, D), lambda i: (0, 0)),
            pl.BlockSpec((1, D), lambda i: (0, 0)),
            pl.BlockSpec((BS2, D), lambda i: (i, 0)),
        ],
        out_specs=pl.BlockSpec((BS2, D), lambda i: (i, 0)),
        out_shape=jax.ShapeDtypeStruct((S, D), jnp.float32),
    )(t, Wproj, bproj.reshape(1, D), x2)

    return out.reshape(B, S, D)


# speedup vs baseline: 1.9391x; 1.2744x over previous
import jax
import jax.numpy as jnp
from jax.experimental import pallas as pl

H = 16


def _msv_kernel(x_ref, wd_ref, wv_ref, bd_ref, bv_ref, t_ref):
    xb = x_ref[...]
    dn = (((1,), (1,)), ((), ()))
    m = jax.lax.dot_general(xb, wd_ref[...], dn,
                            preferred_element_type=jnp.float32) + bd_ref[0]
    v = jax.lax.dot_general(xb, wv_ref[...], dn,
                            preferred_element_type=jnp.float32) + bv_ref[0]
    dh = m.shape[-1] // H
    for h in range(H):
        sl = slice(h * dh, (h + 1) * dh)
        mh = m[:, sl]
        mh = mh - jnp.max(mh, axis=-1, keepdims=True)
        e = jnp.exp(mh)
        t_ref[:, sl] = (e / jnp.sum(e, axis=-1, keepdims=True)) * v[:, sl]


def _proj_kernel(t_ref, w_ref, b_ref, x_ref, o_ref):
    dn = (((1,), (1,)), ((), ()))
    o_ref[...] = (x_ref[...]
                  + jax.lax.dot_general(t_ref[...], w_ref[...], dn,
                                        preferred_element_type=jnp.float32)
                  + b_ref[0])


def kernel(x, Wqkv, bqkv, Wproj, bproj):
    B, S, D = x.shape
    dh = D // H
    scale = dh ** -0.5
    x2 = x.reshape(S, D)
    Wd = (Wqkv[:D] - Wqkv[D:2 * D]) * scale
    Wv = Wqkv[2 * D:]
    bd = ((bqkv[:D] - bqkv[D:2 * D]) * scale).reshape(1, D)
    bv = bqkv[2 * D:].reshape(1, D)

    BS1 = 512
    t = pl.pallas_call(
        _msv_kernel,
        grid=(S // BS1,),
        in_specs=[
            pl.BlockSpec((BS1, D), lambda i: (i, 0)),
            pl.BlockSpec((D, D), lambda i: (0, 0)),
            pl.BlockSpec((D, D), lambda i: (0, 0)),
            pl.BlockSpec((1, D), lambda i: (0, 0)),
            pl.BlockSpec((1, D), lambda i: (0, 0)),
        ],
        out_specs=pl.BlockSpec((BS1, D), lambda i: (i, 0)),
        out_shape=jax.ShapeDtypeStruct((S, D), jnp.float32),
    )(x2, Wd, Wv, bd, bv)

    BS2 = 512
    out = pl.pallas_call(
        _proj_kernel,
        grid=(S // BS2,),
        in_specs=[
            pl.BlockSpec((BS2, D), lambda i: (i, 0)),
            pl.BlockSpec((D, D), lambda i: (0, 0)),
            pl.BlockSpec((1, D), lambda i: (0, 0)),
            pl.BlockSpec((BS2, D), lambda i: (i, 0)),
        ],
        out_specs=pl.BlockSpec((BS2, D), lambda i: (i, 0)),
        out_shape=jax.ShapeDtypeStruct((S, D), jnp.float32),
    )(t, Wproj, bproj.reshape(1, D), x2)

    return out.reshape(B, S, D)


# speedup vs baseline: 2.0812x; 1.0733x over previous
import jax
import jax.numpy as jnp
from jax.experimental import pallas as pl

H = 16


def _msv_kernel(x_ref, wd_ref, wv_ref, bd_ref, bv_ref, t_ref):
    xb = x_ref[...]
    dn = (((1,), (1,)), ((), ()))
    m = jax.lax.dot_general(xb, wd_ref[...], dn,
                            preferred_element_type=jnp.float32) + bd_ref[0]
    v = jax.lax.dot_general(xb, wv_ref[0], dn,
                            preferred_element_type=jnp.float32) + bv_ref[0, 0]
    dh = m.shape[-1] // H
    for h in range(H):
        sl = slice(h * dh, (h + 1) * dh)
        mh = m[:, sl]
        mh = mh - jnp.max(mh, axis=-1, keepdims=True)
        e = jnp.exp(mh)
        t_ref[:, sl] = ((e / jnp.sum(e, axis=-1, keepdims=True))
                        * v[:, sl]).astype(jnp.bfloat16)


def _proj_kernel(t_ref, w_ref, b_ref, x_ref, o_ref):
    dn = (((1,), (1,)), ((), ()))
    o_ref[...] = (x_ref[...]
                  + jax.lax.dot_general(t_ref[...].astype(jnp.float32), w_ref[...],
                                        dn, preferred_element_type=jnp.float32)
                  + b_ref[0])


def kernel(x, Wqkv, bqkv, Wproj, bproj):
    B, S, D = x.shape
    dh = D // H
    scale = dh ** -0.5
    x2 = x.reshape(S, D)
    Wd = (Wqkv[:D] - Wqkv[D:2 * D]) * scale
    bd = ((bqkv[:D] - bqkv[D:2 * D]) * scale).reshape(1, D)
    w3 = Wqkv.reshape(3, D, D)
    b3 = bqkv.reshape(3, 1, D)

    BS1 = 512
    t = pl.pallas_call(
        _msv_kernel,
        grid=(S // BS1,),
        in_specs=[
            pl.BlockSpec((BS1, D), lambda i: (i, 0)),
            pl.BlockSpec((D, D), lambda i: (0, 0)),
            pl.BlockSpec((1, D, D), lambda i: (2, 0, 0)),
            pl.BlockSpec((1, D), lambda i: (0, 0)),
            pl.BlockSpec((1, 1, D), lambda i: (2, 0, 0)),
        ],
        out_specs=pl.BlockSpec((BS1, D), lambda i: (i, 0)),
        out_shape=jax.ShapeDtypeStruct((S, D), jnp.bfloat16),
    )(x2, Wd, w3, bd, b3)

    BS2 = 512
    out = pl.pallas_call(
        _proj_kernel,
        grid=(S // BS2,),
        in_specs=[
            pl.BlockSpec((BS2, D), lambda i: (i, 0)),
            pl.BlockSpec((D, D), lambda i: (0, 0)),
            pl.BlockSpec((1, D), lambda i: (0, 0)),
            pl.BlockSpec((BS2, D), lambda i: (i, 0)),
        ],
        out_specs=pl.BlockSpec((BS2, D), lambda i: (i, 0)),
        out_shape=jax.ShapeDtypeStruct((S, D), jnp.float32),
    )(t, Wproj, bproj.reshape(1, D), x2)

    return out.reshape(B, S, D)
